# baseline (speedup 1.0000x reference)
import functools

import jax
import jax.numpy as jnp
from jax import lax
from jax.experimental import pallas as pl
from jax.experimental.pallas import tpu as pltpu
from jax.experimental.pallas import tpu_sc as plsc

BATCH = 16384
FEATS = 64
NUM_CORES = 2
NUM_SUBCORES = 16
NW = NUM_CORES * NUM_SUBCORES
B_PER_W = BATCH // NW
GROUPS = B_PER_W // 16
GCHUNK = 128
NCHUNK = B_PER_W // GCHUNK
FCHUNK = 16
NFC = FEATS // FCHUNK


def _body(home_hbm, away_hbm, xt_hbm, s_hbm, beta_hbm, mu_hbm, out_hbm,
          hidx_v, aidx_v, sh_v, sa_v, xt_v, beta_v, bsp_v, zidx_v, mu_v,
          out_v, sem, xsem):
    cid = lax.axis_index("c")
    sid = lax.axis_index("s")
    wid = sid * NUM_CORES + cid
    base = wid * B_PER_W

    pltpu.sync_copy(home_hbm.at[pl.ds(base, B_PER_W)], hidx_v)
    pltpu.sync_copy(away_hbm.at[pl.ds(base, B_PER_W)], aidx_v)

    lanes = lax.iota(jnp.int32, 16)
    zidx_v[...] = lanes * 0

    copies = []
    for c in range(NCHUNK):
        sl = pl.ds(c * GCHUNK, GCHUNK)
        copies.append(pltpu.async_copy(s_hbm.at[hidx_v.at[sl]], sh_v.at[sl], sem))
        copies.append(pltpu.async_copy(s_hbm.at[aidx_v.at[sl]], sa_v.at[sl], sem))
    copies.append(pltpu.async_copy(mu_hbm.at[zidx_v], mu_v, sem))
    xcopies = [
        pltpu.async_copy(xt_hbm.at[f, pl.ds(base, B_PER_W)], xt_v.at[f], xsem)
        for f in range(FEATS)
    ]
    pltpu.sync_copy(beta_hbm, beta_v)

    for f in range(1, FEATS):
        bsp_v[f, :] = plsc.load_gather(
            beta_v, [jnp.full((16,), f, dtype=jnp.int32)])
    b0 = beta_v[pl.ds(0, 16)]
    bsp_v[0, :] = plsc.cumsum(jnp.where(lanes == 0, b0, 0.0))

    for cp in copies:
        cp.wait()
    mu_s = mu_v[...]

    zero = jnp.zeros((16,), jnp.float32)
    accs = (zero,) * GROUPS
    for c in range(NFC):
        for k in range(FCHUNK):
            xcopies[c * FCHUNK + k].wait()

        def fbody(f, a):
            bs = bsp_v[f, :]
            return tuple(
                acc + xt_v[f, pl.ds(g * 16, 16)] * bs
                for g, acc in enumerate(a))

        accs = lax.fori_loop(c * FCHUNK, (c + 1) * FCHUNK, fbody, accs)

    for g in range(GROUPS):
        goff = g * 16
        out_v[pl.ds(goff, 16)] = (
            sh_v[pl.ds(goff, 16)] - sa_v[pl.ds(goff, 16)] + mu_s + accs[g])

    pltpu.sync_copy(out_v, out_hbm.at[pl.ds(base, B_PER_W)])


@jax.jit
def kernel(home_idx, away_idx, X, strengths, beta, mu):
    xt = X.T
    run = functools.partial(
        pl.kernel,
        mesh=plsc.VectorSubcoreMesh(core_axis_name="c", subcore_axis_name="s"),
        out_type=jax.ShapeDtypeStruct((BATCH,), jnp.float32),
        compiler_params=pltpu.CompilerParams(needs_layout_passes=False),
        scratch_types=[
            pltpu.VMEM((B_PER_W,), jnp.int32),
            pltpu.VMEM((B_PER_W,), jnp.int32),
            pltpu.VMEM((B_PER_W,), jnp.float32),
            pltpu.VMEM((B_PER_W,), jnp.float32),
            pltpu.VMEM((FEATS, B_PER_W), jnp.float32),
            pltpu.VMEM((FEATS,), jnp.float32),
            pltpu.VMEM((FEATS, 16), jnp.float32),
            pltpu.VMEM((16,), jnp.int32),
            pltpu.VMEM((16,), jnp.float32),
            pltpu.VMEM((B_PER_W,), jnp.float32),
            pltpu.SemaphoreType.DMA,
            pltpu.SemaphoreType.DMA,
        ],
    )(_body)
    return run(home_idx, away_idx, xt, strengths, beta, mu)


# speedup vs baseline: 1.1162x; 1.1162x over previous
import functools

import jax
import jax.numpy as jnp
from jax import lax
from jax.experimental import pallas as pl
from jax.experimental.pallas import tpu as pltpu
from jax.experimental.pallas import tpu_sc as plsc

BATCH = 16384
FEATS = 64
NUM_CORES = 2
NUM_SUBCORES = 16
NW = NUM_CORES * NUM_SUBCORES
B_PER_W = BATCH // NW
GROUPS = B_PER_W // 16
GCHUNK = 128
NCHUNK = B_PER_W // GCHUNK
FCHUNK = 16
NFC = FEATS // FCHUNK


def _body(home_hbm, away_hbm, xt_hbm, s_hbm, beta_hbm, mu_hbm, out_hbm,
          hidx_v, aidx_v, sh_v, sa_v, xt_v, beta_v, bsp_v, zidx_v, mu_v,
          out_v, sem, xsem):
    cid = lax.axis_index("c")
    sid = lax.axis_index("s")
    wid = sid * NUM_CORES + cid
    base = wid * B_PER_W

    pltpu.sync_copy(home_hbm.at[pl.ds(base, B_PER_W)], hidx_v)
    pltpu.sync_copy(away_hbm.at[pl.ds(base, B_PER_W)], aidx_v)

    lanes = lax.iota(jnp.int32, 16)
    zidx_v[...] = lanes * 0

    copies = []
    for c in range(NCHUNK):
        sl = pl.ds(c * GCHUNK, GCHUNK)
        copies.append(pltpu.async_copy(s_hbm.at[hidx_v.at[sl]], sh_v.at[sl], sem))
        copies.append(pltpu.async_copy(s_hbm.at[aidx_v.at[sl]], sa_v.at[sl], sem))
    copies.append(pltpu.async_copy(mu_hbm.at[zidx_v], mu_v, sem))

    @plsc.parallel_loop(0, FEATS)
    def _xfire(f):
        pltpu.async_copy(
            xt_hbm.at[f, pl.ds(base, B_PER_W)], xt_v.at[f], xsem)

    pltpu.sync_copy(beta_hbm, beta_v)

    @plsc.parallel_loop(0, FEATS)
    def _bsplat(f):
        bsp_v[f, :] = plsc.load_gather(
            beta_v, [jnp.broadcast_to(f, (16,)).astype(jnp.int32)])

    for cp in copies:
        cp.wait()
    mu_s = mu_v[...]
    pltpu.make_async_copy(xt_hbm.at[:, pl.ds(base, B_PER_W)], xt_v, xsem).wait()

    zero = jnp.zeros((16,), jnp.float32)

    def fbody(f, a):
        bs = bsp_v[f, :]
        return tuple(
            acc + xt_v[f, pl.ds(g * 16, 16)] * bs
            for g, acc in enumerate(a))

    accs = lax.fori_loop(0, FEATS, fbody, (zero,) * GROUPS)

    for g in range(GROUPS):
        goff = g * 16
        out_v[pl.ds(goff, 16)] = (
            sh_v[pl.ds(goff, 16)] - sa_v[pl.ds(goff, 16)] + mu_s + accs[g])

    pltpu.sync_copy(out_v, out_hbm.at[pl.ds(base, B_PER_W)])


@jax.jit
def kernel(home_idx, away_idx, X, strengths, beta, mu):
    xt = X.T
    run = functools.partial(
        pl.kernel,
        mesh=plsc.VectorSubcoreMesh(core_axis_name="c", subcore_axis_name="s"),
        out_type=jax.ShapeDtypeStruct((BATCH,), jnp.float32),
        compiler_params=pltpu.CompilerParams(needs_layout_passes=False),
        scratch_types=[
            pltpu.VMEM((B_PER_W,), jnp.int32),
            pltpu.VMEM((B_PER_W,), jnp.int32),
            pltpu.VMEM((B_PER_W,), jnp.float32),
            pltpu.VMEM((B_PER_W,), jnp.float32),
            pltpu.VMEM((FEATS, B_PER_W), jnp.float32),
            pltpu.VMEM((FEATS,), jnp.float32),
            pltpu.VMEM((FEATS, 16), jnp.float32),
            pltpu.VMEM((16,), jnp.int32),
            pltpu.VMEM((16,), jnp.float32),
            pltpu.VMEM((B_PER_W,), jnp.float32),
            pltpu.SemaphoreType.DMA,
            pltpu.SemaphoreType.DMA,
        ],
    )(_body)
    return run(home_idx, away_idx, xt, strengths, beta, mu)
